# baseline (speedup 1.0000x reference)
import functools

import jax
import jax.numpy as jnp
from jax import lax
from jax.experimental import pallas as pl
from jax.experimental.pallas import tpu as pltpu
from jax.experimental.pallas import tpu_sc as plsc

B = 1; S = 2048; D = 1024
NH = 16; NKV = 4; HD = 64
E = 8; TOPK = 2; FF = 2048
EPS = 1e-6; THETA = 1000000.0

NEG = -1e30
BF = jnp.bfloat16
F32 = jnp.float32

_TM = 256
_NT = (2 * S + E * _TM) // _TM
_P = _NT * _TM
_NW = 32
_RPW = _P // _NW
_APW = (2 * S) // _NW
_TPW = S // _NW


_TS1 = 512


def _k1_body(x_ref, ln_ref, qkvw_ref, cos_ref, sin_ref, q_ref, k_ref, v_ref):
    x = x_ref[...]
    var = jnp.mean(jnp.square(x), axis=-1, keepdims=True)
    xn = (x * jax.lax.rsqrt(var + EPS)) * ln_ref[...]
    qkv = jnp.dot(xn.astype(BF), qkvw_ref[...].astype(BF),
                  preferred_element_type=F32)
    cos = cos_ref[...]
    sin = sin_ref[...]

    def rope(t, nheads):
        t3 = t.reshape(_TS1, nheads, HD)
        rot = jnp.concatenate([-t3[..., HD // 2:], t3[..., :HD // 2]], axis=-1)
        t2 = t3 * cos[:, None, :] + rot * sin[:, None, :]
        return t2.reshape(_TS1, nheads * HD)

    q = qkv[:, :NH * HD]
    k = qkv[:, NH * HD:(NH + NKV) * HD]
    v = qkv[:, (NH + NKV) * HD:]
    q_ref[...] = rope(q, NH).astype(BF)
    k_ref[...] = rope(k, NKV).astype(BF)
    v_ref[...] = v.astype(BF)


def _qkv_rope(x, ln1_w, qkv_w, cos, sin):
    grid = (S // _TS1,)
    return pl.pallas_call(
        _k1_body,
        grid=grid,
        in_specs=[
            pl.BlockSpec((_TS1, D), lambda t: (t, 0)),
            pl.BlockSpec((1, D), lambda t: (0, 0)),
            pl.BlockSpec((D, (NH + 2 * NKV) * HD), lambda t: (0, 0)),
            pl.BlockSpec((_TS1, HD), lambda t: (t, 0)),
            pl.BlockSpec((_TS1, HD), lambda t: (t, 0)),
        ],
        out_specs=[
            pl.BlockSpec((_TS1, NH * HD), lambda t: (t, 0)),
            pl.BlockSpec((_TS1, NKV * HD), lambda t: (t, 0)),
            pl.BlockSpec((_TS1, NKV * HD), lambda t: (t, 0)),
        ],
        out_shape=[
            jax.ShapeDtypeStruct((S, NH * HD), BF),
            jax.ShapeDtypeStruct((S, NKV * HD), BF),
            jax.ShapeDtypeStruct((S, NKV * HD), BF),
        ],
    )(x, ln1_w.reshape(1, D), qkv_w, cos, sin)


_QB = 256
_REP = NH // NKV


def _k2_body(q_ref, k_ref, v_ref, o_ref, acc_ref, den_ref):
    qb = pl.program_id(0)
    row = jax.lax.broadcasted_iota(jnp.int32, (_QB, _QB), 0)
    col = jax.lax.broadcasted_iota(jnp.int32, (_QB, _QB), 1)
    causal = col <= row
    for h in range(NH):
        kv = h // _REP
        qh = q_ref[:, h * HD:(h + 1) * HD] * BF(HD ** -0.5)

        kc = k_ref[pl.ds(qb * _QB, _QB), kv * HD:(kv + 1) * HD]
        vc = v_ref[pl.ds(qb * _QB, _QB), kv * HD:(kv + 1) * HD]
        s = jax.lax.dot_general(qh, kc, (((1,), (1,)), ((), ())),
                                preferred_element_type=F32)
        e = jnp.exp(jnp.where(causal, s, NEG))
        acc_ref[...] = jnp.dot(e.astype(BF), vc, preferred_element_type=F32)
        den_ref[...] = jnp.sum(e, axis=-1, keepdims=True)

        for j in range(S // _QB - 1):
            @pl.when(j < qb)
            def _(j=j, qh=qh, kv=kv):
                kcj = k_ref[j * _QB:(j + 1) * _QB, kv * HD:(kv + 1) * HD]
                vcj = v_ref[j * _QB:(j + 1) * _QB, kv * HD:(kv + 1) * HD]
                sj = jax.lax.dot_general(qh, kcj, (((1,), (1,)), ((), ())),
                                         preferred_element_type=F32)
                ej = jnp.exp(sj)
                acc_ref[...] += jnp.dot(ej.astype(BF), vcj,
                                        preferred_element_type=F32)
                den_ref[...] += jnp.sum(ej, axis=-1, keepdims=True)

        o_ref[:, h * HD:(h + 1) * HD] = (
            acc_ref[...] * (1.0 / den_ref[...])).astype(BF)


def _attention(q, k, v):
    grid = (S // _QB,)
    return pl.pallas_call(
        _k2_body,
        grid=grid,
        in_specs=[
            pl.BlockSpec((_QB, NH * HD), lambda qb: (qb, 0)),
            pl.BlockSpec((S, NKV * HD), lambda qb: (0, 0)),
            pl.BlockSpec((S, NKV * HD), lambda qb: (0, 0)),
        ],
        out_specs=pl.BlockSpec((_QB, NH * HD), lambda qb: (qb, 0)),
        out_shape=jax.ShapeDtypeStruct((S, NH * HD), BF),
        scratch_shapes=[
            pltpu.VMEM((_QB, HD), F32),
            pltpu.VMEM((_QB, 1), F32),
        ],
    )(q, k, v)


_TS3 = 512


def _k3_body(a_ref, ow_ref, x_ref, h_ref):
    h_ref[...] = x_ref[...] + jnp.dot(a_ref[...], ow_ref[...].astype(BF),
                                      preferred_element_type=F32)


def _oproj_residual(attn, o_w, x):
    grid = (S // _TS3,)
    return pl.pallas_call(
        _k3_body,
        grid=grid,
        in_specs=[
            pl.BlockSpec((_TS3, NH * HD), lambda t: (t, 0)),
            pl.BlockSpec((NH * HD, D), lambda t: (0, 0)),
            pl.BlockSpec((_TS3, D), lambda t: (t, 0)),
        ],
        out_specs=pl.BlockSpec((_TS3, D), lambda t: (t, 0)),
        out_shape=jax.ShapeDtypeStruct((S, D), F32),
    )(attn, o_w, x)


def _k4_body(h_ref, ln_ref, gw_ref, xn_ref, pos_ref, ws_ref, meta_ref):
    h = h_ref[...]
    var = jnp.mean(jnp.square(h), axis=-1, keepdims=True)
    xnb = ((h * jax.lax.rsqrt(var + EPS)) * ln_ref[...]).astype(BF)
    xn_ref[...] = xnb.astype(F32)
    logits = jax.lax.dot_general(xnb, gw_ref[...].astype(BF),
                                 (((1,), (0,)), ((), ())),
                                 preferred_element_type=F32)
    p = jax.nn.softmax(logits, axis=-1)
    idx = jax.lax.broadcasted_iota(jnp.int32, (S, E), 1)
    m0 = jnp.max(p, axis=-1, keepdims=True)
    i0 = jnp.min(jnp.where(p == m0, idx, E), axis=-1, keepdims=True)
    p1m = jnp.where(idx == i0, -1.0, p)
    m1 = jnp.max(p1m, axis=-1, keepdims=True)
    i1 = jnp.min(jnp.where(p1m == m1, idx, E), axis=-1, keepdims=True)
    tot = m0 + m1
    w01 = jnp.concatenate([m0 / tot, m1 / tot], axis=1)
    ws_ref[...] = w01.astype(BF).astype(F32)

    y0 = (idx == i0).astype(F32)
    y1 = (idx == i1).astype(F32)
    tt = y0 + y1

    li = jax.lax.broadcasted_iota(jnp.int32, (S, S), 0)
    lj = jax.lax.broadcasted_iota(jnp.int32, (S, S), 1)
    ltri = (lj < li).astype(BF)
    cexc = jnp.dot(ltri, tt.astype(BF), preferred_element_type=F32)

    counts = jnp.sum(tt, axis=0, keepdims=True)
    rank0 = jnp.sum(cexc * y0, axis=-1, keepdims=True)
    rank1 = jnp.sum((cexc + y0) * y1, axis=-1, keepdims=True)

    ci = counts.astype(jnp.int32)
    pc = ((ci + (_TM - 1)) // _TM) * _TM
    pcf = pc.astype(F32)
    ue = jax.lax.broadcasted_iota(jnp.int32, (E, E), 0)
    uc = jax.lax.broadcasted_iota(jnp.int32, (E, E), 1)
    utri = (ue < uc).astype(BF)
    poff = jnp.dot(pcf.astype(BF), utri, preferred_element_type=F32)

    pos0 = jnp.sum(poff * y0, axis=-1, keepdims=True) + rank0
    pos1 = jnp.sum(poff * y1, axis=-1, keepdims=True) + rank1
    pos_ref[...] = jnp.concatenate([pos0, pos1], axis=1).astype(jnp.int32)

    ends = poff + pcf
    n_act = jnp.sum(pcf, axis=-1, keepdims=True) * (1.0 / _TM)
    it = jax.lax.broadcasted_iota(jnp.int32, (1, 32), 1).astype(F32)
    eidv = jnp.zeros((1, 32), F32)
    for e in range(E):
        eidv = eidv + (it * _TM >= ends[:, e:e + 1]).astype(F32)
    eidv = jnp.minimum(eidv, E - 1)
    eid_last = jnp.sum(jnp.where(it == n_act - 1.0, eidv, 0.0),
                       axis=-1, keepdims=True)
    meta = jnp.where(it < n_act, eidv, eid_last)
    meta = jnp.where(it == 24.0, n_act, meta)
    meta_ref[...] = meta.astype(jnp.int32)


def _router(h, ln2_w, gate_w):
    return pl.pallas_call(
        _k4_body,
        grid=(1,),
        in_specs=[
            pl.BlockSpec((S, D), lambda i: (0, 0)),
            pl.BlockSpec((1, D), lambda i: (0, 0)),
            pl.BlockSpec((D, E), lambda i: (0, 0)),
        ],
        out_specs=[
            pl.BlockSpec((S, D), lambda i: (0, 0)),
            pl.BlockSpec((S, 2), lambda i: (0, 0)),
            pl.BlockSpec((S, 2), lambda i: (0, 0)),
            pl.BlockSpec((1, 32), lambda i: (0, 0)),
        ],
        out_shape=[
            jax.ShapeDtypeStruct((S, D), F32),
            jax.ShapeDtypeStruct((S, 2), jnp.int32),
            jax.ShapeDtypeStruct((S, 2), F32),
            jax.ShapeDtypeStruct((1, 32), jnp.int32),
        ],
    )(h, ln2_w.reshape(1, D), gate_w)


def _dispatch(xn, idx3):
    mesh = plsc.VectorSubcoreMesh(core_axis_name="c", subcore_axis_name="s")

    @functools.partial(
        pl.kernel, mesh=mesh,
        out_type=jax.ShapeDtypeStruct((_P, D), F32),
        scratch_types=[
            pltpu.VMEM((4, 32), jnp.int32),
            pltpu.VMEM((32, D), F32),
            pltpu.SemaphoreType.DMA,
        ])
    def k(xn_hbm, idx_hbm, xs_hbm, posb, rows, sem):
        wid = lax.axis_index("s") * 2 + lax.axis_index("c")
        pltpu.sync_copy(idx_hbm.at[wid], posb)
        for c in range(2):
            tbase = wid * _TPW + c * 32
            pltpu.sync_copy(xn_hbm.at[pl.ds(tbase, 32)], rows)
            for kk in range(2):
                pltpu.async_copy(rows, xs_hbm.at[posb.at[kk * 2 + c]],
                                 sem).wait()

    return k(xn, idx3)


_FB = 512


def _k5_body(meta_ref, xs_ref, w1_ref, w3_ref, w2_ref, ys_ref):
    i = pl.program_id(0)

    @pl.when(i < meta_ref[24])
    def _():
        xs = xs_ref[...]
        acc = jnp.zeros((_TM, D), F32)
        for f in range(FF // _FB):
            w1b = w1_ref[0, :, f * _FB:(f + 1) * _FB]
            w3b = w3_ref[0, :, f * _FB:(f + 1) * _FB]
            w2b = w2_ref[0, f * _FB:(f + 1) * _FB, :]
            a = jnp.dot(xs, w1b, preferred_element_type=F32,
                        precision=jax.lax.Precision.DEFAULT)
            b = jnp.dot(xs, w3b, preferred_element_type=F32,
                        precision=jax.lax.Precision.DEFAULT)
            hh = (a * jax.nn.sigmoid(a)) * b
            acc = acc + jnp.dot(hh, w2b, preferred_element_type=F32,
                                precision=jax.lax.Precision.DEFAULT)
        ys_ref[...] = acc.astype(BF).astype(F32)


def _grouped_ffn(meta, xs, w1, w3, w2):
    grid_spec = pltpu.PrefetchScalarGridSpec(
        num_scalar_prefetch=1,
        grid=(_NT,),
        in_specs=[
            pl.BlockSpec((_TM, D), lambda i, m: (i, 0)),
            pl.BlockSpec((1, D, FF), lambda i, m: (m[i], 0, 0)),
            pl.BlockSpec((1, D, FF), lambda i, m: (m[i], 0, 0)),
            pl.BlockSpec((1, FF, D), lambda i, m: (m[i], 0, 0)),
        ],
        out_specs=pl.BlockSpec((_TM, D), lambda i, m: (i, 0)),
    )
    return pl.pallas_call(
        _k5_body,
        grid_spec=grid_spec,
        out_shape=jax.ShapeDtypeStruct((_P, D), F32),
        compiler_params=pltpu.CompilerParams(
            vmem_limit_bytes=112 * 1024 * 1024),
    )(meta, xs, w1, w3, w2)


def _combine(h, ys, pos, ws):
    mesh = plsc.VectorSubcoreMesh(core_axis_name="c", subcore_axis_name="s")

    @functools.partial(
        pl.kernel, mesh=mesh,
        out_type=jax.ShapeDtypeStruct((S, D), F32),
        scratch_types=[
            pltpu.VMEM((_APW,), jnp.int32),
            pltpu.VMEM((_APW + 16,), F32),
            pltpu.VMEM((64, D), F32),
            pltpu.VMEM((32, D), F32),
            pltpu.SemaphoreType.DMA,
        ])
    def k(h_hbm, ys_hbm, pos_hbm, ws_hbm, out_hbm, posb, wsb, rows, hb, sem):
        wid = lax.axis_index("s") * 2 + lax.axis_index("c")
        pltpu.sync_copy(pos_hbm.at[pl.ds(wid * _APW, _APW)], posb)
        pltpu.sync_copy(ws_hbm.at[pl.ds(wid * _APW, _APW)],
                        wsb.at[pl.ds(0, _APW)])
        for c in range(_TPW // 32):
            tbase = wid * _TPW + c * 32
            pltpu.sync_copy(h_hbm.at[pl.ds(tbase, 32)], hb)
            idx_slice = posb.at[pl.ds(c * 64, 64)]
            pltpu.async_copy(ys_hbm.at[idx_slice], rows, sem).wait()

            def tbody(j, carry):
                wv = wsb[pl.ds(c * 64 + 2 * j, 16)]
                w0 = wv[0]
                w1v = wv[1]

                def vbody(u, c2):
                    sl = pl.ds(u * 16, 16)
                    hb[j, sl] = (hb[j, sl] + w0 * rows[2 * j, sl]
                                 + w1v * rows[2 * j + 1, sl])
                    return c2

                return lax.fori_loop(0, D // 16, vbody, carry)

            lax.fori_loop(0, 32, tbody, 0)
            pltpu.sync_copy(hb, out_hbm.at[pl.ds(tbase, 32)])

    return k(h, ys, pos, ws)


def kernel(hidden_states, attention_mask, position_ids, ln1_w, q_w, k_w, v_w,
           o_w, ln2_w, gate_w, w1, w3, w2):
    del attention_mask
    x = hidden_states.reshape(S, D)
    pos_ids = position_ids.reshape(S).astype(F32)

    inv = 1.0 / (THETA ** (jnp.arange(0, HD, 2, dtype=F32) / HD))
    ang = pos_ids[:, None] * inv[None, :]
    cos = jnp.concatenate([jnp.cos(ang), jnp.cos(ang)], axis=-1)
    sin = jnp.concatenate([jnp.sin(ang), jnp.sin(ang)], axis=-1)

    qkv_w = jnp.concatenate([q_w, k_w, v_w], axis=1)
    q, k, v = _qkv_rope(x, ln1_w, qkv_w, cos, sin)
    attn = _attention(q, k, v)
    h = _oproj_residual(attn, o_w, x)
    xn2, pos01, ws01, meta = _router(h, ln2_w, gate_w)
    idx3 = jnp.concatenate([pos01[:, 0].reshape(_NW, 2, _TPW // 2),
                            pos01[:, 1].reshape(_NW, 2, _TPW // 2)], axis=1)
    xs = _dispatch(xn2, idx3)
    ys = _grouped_ffn(meta.reshape(32), xs, w1, w3, w2)
    out = _combine(h, ys, pos01.reshape(2 * S), ws01.reshape(2 * S))
    return out.reshape(B, S, D)


# speedup vs baseline: 1.3802x; 1.3802x over previous
import functools

import jax
import jax.numpy as jnp
from jax import lax
from jax.experimental import pallas as pl
from jax.experimental.pallas import tpu as pltpu
from jax.experimental.pallas import tpu_sc as plsc

B = 1; S = 2048; D = 1024
NH = 16; NKV = 4; HD = 64
E = 8; TOPK = 2; FF = 2048
EPS = 1e-6; THETA = 1000000.0

NEG = -1e30
BF = jnp.bfloat16
F32 = jnp.float32

_TM = 256
_NT = (2 * S + E * _TM) // _TM
_P = _NT * _TM
_NW = 32
_RPW = _P // _NW
_APW = (2 * S) // _NW
_TPW = S // _NW


_TS1 = 512


def _k1_body(x_ref, ln_ref, qkvw_ref, cos_ref, sin_ref, q_ref, k_ref, v_ref):
    x = x_ref[...]
    var = jnp.mean(jnp.square(x), axis=-1, keepdims=True)
    xn = (x * jax.lax.rsqrt(var + EPS)) * ln_ref[...]
    qkv = jnp.dot(xn.astype(BF), qkvw_ref[...].astype(BF),
                  preferred_element_type=F32)
    cos = cos_ref[...]
    sin = sin_ref[...]

    def rope(t, nheads):
        t3 = t.reshape(_TS1, nheads, HD)
        rot = jnp.concatenate([-t3[..., HD // 2:], t3[..., :HD // 2]], axis=-1)
        t2 = t3 * cos[:, None, :] + rot * sin[:, None, :]
        return t2.reshape(_TS1, nheads * HD)

    q = qkv[:, :NH * HD]
    k = qkv[:, NH * HD:(NH + NKV) * HD]
    v = qkv[:, (NH + NKV) * HD:]
    q_ref[...] = rope(q, NH).astype(BF)
    k_ref[...] = rope(k, NKV).astype(BF)
    v_ref[...] = v.astype(BF)


def _qkv_rope(x, ln1_w, qkv_w, cos, sin):
    grid = (S // _TS1,)
    return pl.pallas_call(
        _k1_body,
        grid=grid,
        in_specs=[
            pl.BlockSpec((_TS1, D), lambda t: (t, 0)),
            pl.BlockSpec((1, D), lambda t: (0, 0)),
            pl.BlockSpec((D, (NH + 2 * NKV) * HD), lambda t: (0, 0)),
            pl.BlockSpec((_TS1, HD), lambda t: (t, 0)),
            pl.BlockSpec((_TS1, HD), lambda t: (t, 0)),
        ],
        out_specs=[
            pl.BlockSpec((_TS1, NH * HD), lambda t: (t, 0)),
            pl.BlockSpec((_TS1, NKV * HD), lambda t: (t, 0)),
            pl.BlockSpec((_TS1, NKV * HD), lambda t: (t, 0)),
        ],
        out_shape=[
            jax.ShapeDtypeStruct((S, NH * HD), BF),
            jax.ShapeDtypeStruct((S, NKV * HD), BF),
            jax.ShapeDtypeStruct((S, NKV * HD), BF),
        ],
    )(x, ln1_w.reshape(1, D), qkv_w, cos, sin)


_QB = 256
_REP = NH // NKV


def _k2_body(q_ref, k_ref, v_ref, o_ref):
    qb = pl.program_id(0)
    row = jax.lax.broadcasted_iota(jnp.int32, (_QB, S), 0) + qb * _QB
    col = jax.lax.broadcasted_iota(jnp.int32, (_QB, S), 1)
    causal = col <= row
    outs = []
    for h in range(NH):
        kv = h // _REP
        qh = q_ref[:, h * HD:(h + 1) * HD] * BF(HD ** -0.5)
        kh = k_ref[:, kv * HD:(kv + 1) * HD]
        vh = v_ref[:, kv * HD:(kv + 1) * HD]
        s = jax.lax.dot_general(qh, kh, (((1,), (1,)), ((), ())),
                                preferred_element_type=F32)
        e = jnp.exp(jnp.where(causal, s, NEG))
        p = e * (1.0 / jnp.sum(e, axis=-1, keepdims=True))
        outs.append(jnp.dot(p.astype(BF), vh, preferred_element_type=F32))
    o_ref[...] = jnp.concatenate(outs, axis=1).astype(BF)


def _attention(q, k, v):
    grid = (S // _QB,)
    return pl.pallas_call(
        _k2_body,
        grid=grid,
        in_specs=[
            pl.BlockSpec((_QB, NH * HD), lambda qb: (qb, 0)),
            pl.BlockSpec((S, NKV * HD), lambda qb: (0, 0)),
            pl.BlockSpec((S, NKV * HD), lambda qb: (0, 0)),
        ],
        out_specs=pl.BlockSpec((_QB, NH * HD), lambda qb: (qb, 0)),
        out_shape=jax.ShapeDtypeStruct((S, NH * HD), BF),
    )(q, k, v)


_TS3 = 512


def _k3_body(a_ref, ow_ref, x_ref, h_ref):
    h_ref[...] = x_ref[...] + jnp.dot(a_ref[...], ow_ref[...].astype(BF),
                                      preferred_element_type=F32)


def _oproj_residual(attn, o_w, x):
    grid = (S // _TS3,)
    return pl.pallas_call(
        _k3_body,
        grid=grid,
        in_specs=[
            pl.BlockSpec((_TS3, NH * HD), lambda t: (t, 0)),
            pl.BlockSpec((NH * HD, D), lambda t: (0, 0)),
            pl.BlockSpec((_TS3, D), lambda t: (t, 0)),
        ],
        out_specs=pl.BlockSpec((_TS3, D), lambda t: (t, 0)),
        out_shape=jax.ShapeDtypeStruct((S, D), F32),
    )(attn, o_w, x)


def _k4_body(h_ref, ln_ref, gw_ref, xn_ref, pos_ref, ws_ref, meta_ref):
    h = h_ref[...]
    var = jnp.mean(jnp.square(h), axis=-1, keepdims=True)
    xnb = ((h * jax.lax.rsqrt(var + EPS)) * ln_ref[...]).astype(BF)
    xn_ref[...] = xnb.astype(F32)
    logits = jax.lax.dot_general(xnb, gw_ref[...].astype(BF),
                                 (((1,), (0,)), ((), ())),
                                 preferred_element_type=F32)
    p = jax.nn.softmax(logits, axis=-1)
    idx = jax.lax.broadcasted_iota(jnp.int32, (S, E), 1)
    m0 = jnp.max(p, axis=-1, keepdims=True)
    i0 = jnp.min(jnp.where(p == m0, idx, E), axis=-1, keepdims=True)
    p1m = jnp.where(idx == i0, -1.0, p)
    m1 = jnp.max(p1m, axis=-1, keepdims=True)
    i1 = jnp.min(jnp.where(p1m == m1, idx, E), axis=-1, keepdims=True)
    tot = m0 + m1
    w01 = jnp.concatenate([m0 / tot, m1 / tot], axis=1)
    ws_ref[...] = w01.astype(BF).astype(F32)

    y0 = (idx == i0).astype(F32)
    y1 = (idx == i1).astype(F32)
    tt = y0 + y1

    li = jax.lax.broadcasted_iota(jnp.int32, (S, S), 0)
    lj = jax.lax.broadcasted_iota(jnp.int32, (S, S), 1)
    ltri = (lj < li).astype(BF)
    cexc = jnp.dot(ltri, tt.astype(BF), preferred_element_type=F32)

    counts = jnp.sum(tt, axis=0, keepdims=True)
    rank0 = jnp.sum(cexc * y0, axis=-1, keepdims=True)
    rank1 = jnp.sum((cexc + y0) * y1, axis=-1, keepdims=True)

    ci = counts.astype(jnp.int32)
    pc = ((ci + (_TM - 1)) // _TM) * _TM
    pcf = pc.astype(F32)
    ue = jax.lax.broadcasted_iota(jnp.int32, (E, E), 0)
    uc = jax.lax.broadcasted_iota(jnp.int32, (E, E), 1)
    utri = (ue < uc).astype(BF)
    poff = jnp.dot(pcf.astype(BF), utri, preferred_element_type=F32)

    pos0 = jnp.sum(poff * y0, axis=-1, keepdims=True) + rank0
    pos1 = jnp.sum(poff * y1, axis=-1, keepdims=True) + rank1
    pos_ref[...] = jnp.concatenate([pos0, pos1], axis=1).astype(jnp.int32)

    ends = poff + pcf
    n_act = jnp.sum(pcf, axis=-1, keepdims=True) * (1.0 / _TM)
    it = jax.lax.broadcasted_iota(jnp.int32, (1, 32), 1).astype(F32)
    eidv = jnp.zeros((1, 32), F32)
    for e in range(E):
        eidv = eidv + (it * _TM >= ends[:, e:e + 1]).astype(F32)
    eidv = jnp.minimum(eidv, E - 1)
    eid_last = jnp.sum(jnp.where(it == n_act - 1.0, eidv, 0.0),
                       axis=-1, keepdims=True)
    meta = jnp.where(it < n_act, eidv, eid_last)
    meta = jnp.where(it == 24.0, n_act, meta)
    meta_ref[...] = meta.astype(jnp.int32)


def _router(h, ln2_w, gate_w):
    return pl.pallas_call(
        _k4_body,
        grid=(1,),
        in_specs=[
            pl.BlockSpec((S, D), lambda i: (0, 0)),
            pl.BlockSpec((1, D), lambda i: (0, 0)),
            pl.BlockSpec((D, E), lambda i: (0, 0)),
        ],
        out_specs=[
            pl.BlockSpec((S, D), lambda i: (0, 0)),
            pl.BlockSpec((S, 2), lambda i: (0, 0)),
            pl.BlockSpec((S, 2), lambda i: (0, 0)),
            pl.BlockSpec((1, 32), lambda i: (0, 0)),
        ],
        out_shape=[
            jax.ShapeDtypeStruct((S, D), F32),
            jax.ShapeDtypeStruct((S, 2), jnp.int32),
            jax.ShapeDtypeStruct((S, 2), F32),
            jax.ShapeDtypeStruct((1, 32), jnp.int32),
        ],
    )(h, ln2_w.reshape(1, D), gate_w)


def _dispatch(xn, idx3):
    mesh = plsc.VectorSubcoreMesh(core_axis_name="c", subcore_axis_name="s")

    @functools.partial(
        pl.kernel, mesh=mesh,
        out_type=jax.ShapeDtypeStruct((_P, D), F32),
        scratch_types=[
            pltpu.VMEM((4, 32), jnp.int32),
            pltpu.VMEM((32, D), F32),
            pltpu.SemaphoreType.DMA,
        ])
    def k(xn_hbm, idx_hbm, xs_hbm, posb, rows, sem):
        wid = lax.axis_index("s") * 2 + lax.axis_index("c")
        pltpu.sync_copy(idx_hbm.at[wid], posb)
        for c in range(2):
            tbase = wid * _TPW + c * 32
            pltpu.sync_copy(xn_hbm.at[pl.ds(tbase, 32)], rows)
            for kk in range(2):
                pltpu.async_copy(rows, xs_hbm.at[posb.at[kk * 2 + c]],
                                 sem).wait()

    return k(xn, idx3)


_FB = 512


def _k5_body(meta_ref, xs_ref, w1_ref, w3_ref, w2_ref, ys_ref):
    i = pl.program_id(0)

    @pl.when(i < meta_ref[24])
    def _():
        xs = xs_ref[...]
        acc = jnp.zeros((_TM, D), F32)
        for f in range(FF // _FB):
            w1b = w1_ref[0, :, f * _FB:(f + 1) * _FB]
            w3b = w3_ref[0, :, f * _FB:(f + 1) * _FB]
            w2b = w2_ref[0, f * _FB:(f + 1) * _FB, :]
            a = jnp.dot(xs, w1b, preferred_element_type=F32,
                        precision=jax.lax.Precision.DEFAULT)
            b = jnp.dot(xs, w3b, preferred_element_type=F32,
                        precision=jax.lax.Precision.DEFAULT)
            hh = (a * jax.nn.sigmoid(a)) * b
            acc = acc + jnp.dot(hh, w2b, preferred_element_type=F32,
                                precision=jax.lax.Precision.DEFAULT)
        ys_ref[...] = acc.astype(BF).astype(F32)


def _grouped_ffn(meta, xs, w1, w3, w2):
    grid_spec = pltpu.PrefetchScalarGridSpec(
        num_scalar_prefetch=1,
        grid=(_NT,),
        in_specs=[
            pl.BlockSpec((_TM, D), lambda i, m: (i, 0)),
            pl.BlockSpec((1, D, FF), lambda i, m: (m[i], 0, 0)),
            pl.BlockSpec((1, D, FF), lambda i, m: (m[i], 0, 0)),
            pl.BlockSpec((1, FF, D), lambda i, m: (m[i], 0, 0)),
        ],
        out_specs=pl.BlockSpec((_TM, D), lambda i, m: (i, 0)),
    )
    return pl.pallas_call(
        _k5_body,
        grid_spec=grid_spec,
        out_shape=jax.ShapeDtypeStruct((_P, D), F32),
        compiler_params=pltpu.CompilerParams(
            vmem_limit_bytes=112 * 1024 * 1024),
    )(meta, xs, w1, w3, w2)


def _combine(h, ys, pos, ws):
    mesh = plsc.VectorSubcoreMesh(core_axis_name="c", subcore_axis_name="s")

    @functools.partial(
        pl.kernel, mesh=mesh,
        out_type=jax.ShapeDtypeStruct((S, D), F32),
        scratch_types=[
            pltpu.VMEM((_APW,), jnp.int32),
            pltpu.VMEM((_APW + 16,), F32),
            pltpu.VMEM((64, D), F32),
            pltpu.VMEM((32, D), F32),
            pltpu.SemaphoreType.DMA,
        ])
    def k(h_hbm, ys_hbm, pos_hbm, ws_hbm, out_hbm, posb, wsb, rows, hb, sem):
        wid = lax.axis_index("s") * 2 + lax.axis_index("c")
        pltpu.sync_copy(pos_hbm.at[pl.ds(wid * _APW, _APW)], posb)
        pltpu.sync_copy(ws_hbm.at[pl.ds(wid * _APW, _APW)],
                        wsb.at[pl.ds(0, _APW)])
        for c in range(_TPW // 32):
            tbase = wid * _TPW + c * 32
            pltpu.sync_copy(h_hbm.at[pl.ds(tbase, 32)], hb)
            idx_slice = posb.at[pl.ds(c * 64, 64)]
            pltpu.async_copy(ys_hbm.at[idx_slice], rows, sem).wait()

            def tbody(j, carry):
                wv = wsb[pl.ds(c * 64 + 2 * j, 16)]
                w0 = wv[0]
                w1v = wv[1]

                def vbody(u, c2):
                    sl = pl.ds(u * 16, 16)
                    hb[j, sl] = (hb[j, sl] + w0 * rows[2 * j, sl]
                                 + w1v * rows[2 * j + 1, sl])
                    return c2

                return lax.fori_loop(0, D // 16, vbody, carry)

            lax.fori_loop(0, 32, tbody, 0)
            pltpu.sync_copy(hb, out_hbm.at[pl.ds(tbase, 32)])

    return k(h, ys, pos, ws)


def kernel(hidden_states, attention_mask, position_ids, ln1_w, q_w, k_w, v_w,
           o_w, ln2_w, gate_w, w1, w3, w2):
    del attention_mask
    x = hidden_states.reshape(S, D)
    pos_ids = position_ids.reshape(S).astype(F32)

    inv = 1.0 / (THETA ** (jnp.arange(0, HD, 2, dtype=F32) / HD))
    ang = pos_ids[:, None] * inv[None, :]
    cos = jnp.concatenate([jnp.cos(ang), jnp.cos(ang)], axis=-1)
    sin = jnp.concatenate([jnp.sin(ang), jnp.sin(ang)], axis=-1)

    qkv_w = jnp.concatenate([q_w, k_w, v_w], axis=1)
    q, k, v = _qkv_rope(x, ln1_w, qkv_w, cos, sin)
    attn = _attention(q, k, v)
    h = _oproj_residual(attn, o_w, x)
    xn2, pos01, ws01, meta = _router(h, ln2_w, gate_w)
    idx3 = jnp.concatenate([pos01[:, 0].reshape(_NW, 2, _TPW // 2),
                            pos01[:, 1].reshape(_NW, 2, _TPW // 2)], axis=1)
    xs = _dispatch(xn2, idx3)
    ys = _grouped_ffn(meta.reshape(32), xs, w1, w3, w2)
    out = _combine(h, ys, pos01.reshape(2 * S), ws01.reshape(2 * S))
    return out.reshape(B, S, D)
